# baseline (speedup 1.0000x reference)
import jax
import jax.numpy as jnp
from jax import lax
from jax.experimental import pallas as pl
from jax.experimental.pallas import tpu as pltpu
from jax.experimental.pallas import tpu_sc as plsc

D = 64
V = 1000000
SCALE = 8.0

NUM_CORES = 2
NUM_SUBCORES = 16
NW = NUM_CORES * NUM_SUBCORES

T_FULL = V // 128
K1_ITERS = (T_FULL + 1 + NW - 1) // NW


def _k1_body(lut_t, o_hbm, in0, in1, out0, out1, tailbuf, tailout,
             isem0, isem1, osem0, osem1):
    wid = lax.axis_index("s") * NUM_CORES + lax.axis_index("c")
    ins = (in0, in1)
    outs = (out0, out1)
    isems = (isem0, isem1)
    osems = (osem0, osem1)
    iota = lax.iota(jnp.int32, 16)

    def in_start(t, buf, sem):
        pltpu.async_copy(lut_t.at[:, pl.ds(t * 128, 128)], buf, sem)

    def in_wait(buf, sem):
        pltpu.make_async_copy(lut_t.at[:, pl.ds(0, 128)], buf, sem).wait()

    def out_start(t, buf, sem):
        pltpu.async_copy(buf, o_hbm.at[pl.ds(t * 64, 64)], sem)

    def out_wait(buf, sem):
        pltpu.make_async_copy(buf, o_hbm.at[pl.ds(0, 64)], sem).wait()

    def transform(src, dst, nrows):
        def jbody(j, carry):
            c0 = jnp.broadcast_to(2 * j, (16,)).astype(jnp.int32)
            c1 = c0 + 1
            for k in range(8):
                row_idx = iota + 16 * (k % 4)
                col = c0 if k < 4 else c1
                v = plsc.load_gather(src, [row_idx, col])
                dst[j, pl.ds(16 * k, 16)] = v * SCALE
            return carry

        lax.fori_loop(0, nrows, jbody, 0)

    @pl.when(wid < T_FULL)
    def _():
        in_start(wid, in0, isem0)

    def loop(i, carry):
        t = wid + NW * i
        b = lax.rem(i, 2)
        t_next = t + NW

        @pl.when(t < T_FULL)
        def _():
            for b_s in range(2):
                @pl.when(b == b_s)
                def _():
                    @pl.when(t_next < T_FULL)
                    def _():
                        in_start(t_next, ins[1 - b_s], isems[1 - b_s])

                    in_wait(ins[b_s], isems[b_s])

                    @pl.when(i >= 2)
                    def _():
                        out_wait(outs[b_s], osems[b_s])

                    transform(ins[b_s], outs[b_s], 64)
                    out_start(t, outs[b_s], osems[b_s])

        @pl.when(t == T_FULL)
        def _():
            pltpu.sync_copy(lut_t.at[:, pl.ds(T_FULL * 128, 64)], tailbuf)
            transform(tailbuf, tailout, 32)
            pltpu.sync_copy(tailout, o_hbm.at[pl.ds(T_FULL * 64, 32)])

        return carry

    lax.fori_loop(0, K1_ITERS, loop, 0)

    n_full = (T_FULL - wid + NW - 1) // NW

    @pl.when(n_full >= 1)
    def _():
        blast = lax.rem(n_full - 1, 2)
        for b_s in range(2):
            @pl.when(blast == b_s)
            def _():
                out_wait(outs[b_s], osems[b_s])

    @pl.when(n_full >= 2)
    def _():
        bprev = lax.rem(n_full - 2, 2)
        for b_s in range(2):
            @pl.when(bprev == b_s)
            def _():
                out_wait(outs[b_s], osems[b_s])


K2_UNITS_PER_W = 3200 // NW


def _k2_body(xt, lut_lin, out5, i0, i1, r0, r1, ob0, ob1,
             xsem0, xsem1, gsem0, gsem1, osem0, osem1):
    wid = lax.axis_index("s") * NUM_CORES + lax.axis_index("c")
    u0 = wid * K2_UNITS_PER_W
    ibufs = (i0, i1)
    rows = (r0, r1)
    obufs = (ob0, ob1)
    xsems = (xsem0, xsem1)
    gsems = (gsem0, gsem1)
    osems = (osem0, osem1)
    iota = lax.iota(jnp.int32, 16)

    def idx_start(u, buf, sem):
        s = u // 16
        q = lax.rem(u, 16)
        pltpu.async_copy(xt.at[s, pl.ds(q * 256, 256)], buf, sem)

    def idx_wait(buf, sem):
        pltpu.make_async_copy(xt.at[0, pl.ds(0, 256)], buf, sem).wait()

    def gather_start(ibuf, rbuf, sem):
        pltpu.async_copy(lut_lin.at[ibuf], rbuf, sem)

    def gather_wait(ibuf, rbuf, sem):
        pltpu.make_async_copy(lut_lin.at[ibuf], rbuf, sem).wait()

    def out_start(u, obuf, sem):
        s = u // 16
        q = lax.rem(u, 16)
        for g in range(8):
            pltpu.async_copy(obuf.at[g], out5.at[s, g, pl.ds(q * 2, 2)], sem)

    def out_wait(obuf, sem):
        for g in range(8):
            pltpu.make_async_copy(obuf.at[g], out5.at[0, g, pl.ds(0, 2)], sem).wait()

    def transform(rbuf, obuf):
        def gbody(g, carry):
            for cc in range(8):
                col = jnp.broadcast_to(8 * g + cc, (16,)).astype(jnp.int32)
                for t2 in range(2):
                    for m in range(8):
                        row_idx = iota + (128 * t2 + 16 * m)
                        v = plsc.load_gather(rbuf, [row_idx, col])
                        obuf[g, t2, cc, pl.ds(16 * m, 16)] = v
            return carry

        lax.fori_loop(0, 8, gbody, 0)

    pltpu.sync_copy(xt.at[u0 // 16, pl.ds(lax.rem(u0, 16) * 256, 256)], i0)
    gather_start(i0, r0, gsem0)
    idx_start(u0 + 1, i1, xsem1)

    def loop(i, carry):
        u = u0 + i
        b = lax.rem(i, 2)
        for b_s in range(2):
            @pl.when(b == b_s)
            def _():
                ib, rb, ob = ibufs[b_s], rows[b_s], obufs[b_s]
                nib, nrb = ibufs[1 - b_s], rows[1 - b_s]

                @pl.when(i + 1 < K2_UNITS_PER_W)
                def _():
                    idx_wait(nib, xsems[1 - b_s])
                    gather_start(nib, nrb, gsems[1 - b_s])

                gather_wait(ib, rb, gsems[b_s])

                @pl.when(i + 2 < K2_UNITS_PER_W)
                def _():
                    idx_start(u + 2, ib, xsems[b_s])

                @pl.when(i >= 2)
                def _():
                    out_wait(ob, osems[b_s])

                transform(rb, ob)
                out_start(u, ob, osems[b_s])
        return carry

    lax.fori_loop(0, K2_UNITS_PER_W, loop, 0)

    for i_tail in (K2_UNITS_PER_W - 2, K2_UNITS_PER_W - 1):
        b_t = i_tail % 2
        out_wait(obufs[b_t], osems[b_t])


@jax.jit
def _run(xt, lut_t):
    mesh = plsc.VectorSubcoreMesh(core_axis_name="c", subcore_axis_name="s")
    k1 = pl.kernel(
        _k1_body,
        out_type=jax.ShapeDtypeStruct((V // 2, 128), jnp.float32),
        mesh=mesh,
        compiler_params=pltpu.CompilerParams(
            use_tc_tiling_on_sc=True, needs_layout_passes=False
        ),
        scratch_types=[
            pltpu.VMEM((64, 128), jnp.float32),
            pltpu.VMEM((64, 128), jnp.float32),
            pltpu.VMEM((64, 128), jnp.float32),
            pltpu.VMEM((64, 128), jnp.float32),
            pltpu.VMEM((64, 64), jnp.float32),
            pltpu.VMEM((32, 128), jnp.float32),
            pltpu.SemaphoreType.DMA,
            pltpu.SemaphoreType.DMA,
            pltpu.SemaphoreType.DMA,
            pltpu.SemaphoreType.DMA,
        ],
    )
    o = k1(lut_t)
    lut_lin = o.reshape(V, D)
    k2 = pl.kernel(
        _k2_body,
        out_type=jax.ShapeDtypeStruct((200, 8, 32, 8, 128), jnp.float32),
        mesh=mesh,
        compiler_params=pltpu.CompilerParams(
            use_tc_tiling_on_sc=False, needs_layout_passes=False
        ),
        scratch_types=[
            pltpu.VMEM((256,), jnp.int32),
            pltpu.VMEM((256,), jnp.int32),
            pltpu.VMEM((256, 64), jnp.float32),
            pltpu.VMEM((256, 64), jnp.float32),
            pltpu.VMEM((8, 2, 8, 128), jnp.float32),
            pltpu.VMEM((8, 2, 8, 128), jnp.float32),
            pltpu.SemaphoreType.DMA,
            pltpu.SemaphoreType.DMA,
            pltpu.SemaphoreType.DMA,
            pltpu.SemaphoreType.DMA,
            pltpu.SemaphoreType.DMA,
            pltpu.SemaphoreType.DMA,
        ],
    )
    return k2(xt, lut_lin)


def kernel(x, lut):
    out5 = _run(x.T, lut.T)
    return out5.transpose(2, 4, 0, 1, 3).reshape(4096, 200, 64)


# speedup vs baseline: 2.0278x; 2.0278x over previous
import jax
import jax.numpy as jnp
from jax import lax
from jax.experimental import pallas as pl
from jax.experimental.pallas import tpu as pltpu
from jax.experimental.pallas import tpu_sc as plsc

D = 64
V = 1000000
SCALE = 8.0

NUM_CORES = 2
NUM_SUBCORES = 16
NW = NUM_CORES * NUM_SUBCORES

T_FULL = V // 128
K1_ITERS = (T_FULL + 1 + NW - 1) // NW


def _k1_body(lut_t, o_hbm, in0, in1, out0, out1, tailbuf, tailout,
             isem0, isem1, osem0, osem1):
    wid = lax.axis_index("s") * NUM_CORES + lax.axis_index("c")
    ins = (in0, in1)
    outs = (out0, out1)
    isems = (isem0, isem1)
    osems = (osem0, osem1)
    iota = lax.iota(jnp.int32, 16)

    def in_start(t, buf, sem):
        pltpu.async_copy(lut_t.at[:, pl.ds(t * 128, 128)], buf, sem)

    def in_wait(buf, sem):
        pltpu.make_async_copy(lut_t.at[:, pl.ds(0, 128)], buf, sem).wait()

    def out_start(t, buf, sem):
        pltpu.async_copy(buf, o_hbm.at[pl.ds(t * 64, 64)], sem)

    def out_wait(buf, sem):
        pltpu.make_async_copy(buf, o_hbm.at[pl.ds(0, 64)], sem).wait()

    def transform(src, dst, nrows):
        @plsc.parallel_loop(0, nrows, unroll=4)
        def _(j):
            c0 = jnp.broadcast_to(2 * j, (16,)).astype(jnp.int32)
            c1 = c0 + 1
            for k in range(8):
                row_idx = iota + 16 * (k % 4)
                col = c0 if k < 4 else c1
                v = plsc.load_gather(src, [row_idx, col])
                dst[j, pl.ds(16 * k, 16)] = v * SCALE

    @pl.when(wid < T_FULL)
    def _():
        in_start(wid, in0, isem0)

    def loop(i, carry):
        t = wid + NW * i
        b = lax.rem(i, 2)
        t_next = t + NW

        @pl.when(t < T_FULL)
        def _():
            for b_s in range(2):
                @pl.when(b == b_s)
                def _():
                    @pl.when(t_next < T_FULL)
                    def _():
                        in_start(t_next, ins[1 - b_s], isems[1 - b_s])

                    in_wait(ins[b_s], isems[b_s])

                    @pl.when(i >= 2)
                    def _():
                        out_wait(outs[b_s], osems[b_s])

                    transform(ins[b_s], outs[b_s], 64)
                    out_start(t, outs[b_s], osems[b_s])

        @pl.when(t == T_FULL)
        def _():
            pltpu.sync_copy(lut_t.at[:, pl.ds(T_FULL * 128, 64)], tailbuf)
            transform(tailbuf, tailout, 32)
            pltpu.sync_copy(tailout, o_hbm.at[pl.ds(T_FULL * 64, 32)])

        return carry

    lax.fori_loop(0, K1_ITERS, loop, 0)

    n_full = (T_FULL - wid + NW - 1) // NW

    @pl.when(n_full >= 1)
    def _():
        blast = lax.rem(n_full - 1, 2)
        for b_s in range(2):
            @pl.when(blast == b_s)
            def _():
                out_wait(outs[b_s], osems[b_s])

    @pl.when(n_full >= 2)
    def _():
        bprev = lax.rem(n_full - 2, 2)
        for b_s in range(2):
            @pl.when(bprev == b_s)
            def _():
                out_wait(outs[b_s], osems[b_s])


K2_UNITS_PER_W = 3200 // NW


def _k2_body(xt, lut_lin, out5, i0, i1, r0, r1, ob0, ob1,
             xsem0, xsem1, gsem0, gsem1, osem0, osem1):
    wid = lax.axis_index("s") * NUM_CORES + lax.axis_index("c")
    u0 = wid * K2_UNITS_PER_W
    ibufs = (i0, i1)
    rows = (r0, r1)
    obufs = (ob0, ob1)
    xsems = (xsem0, xsem1)
    gsems = (gsem0, gsem1)
    osems = (osem0, osem1)
    iota = lax.iota(jnp.int32, 16)

    def idx_start(u, buf, sem):
        s = u // 16
        q = lax.rem(u, 16)
        pltpu.async_copy(xt.at[s, pl.ds(q * 256, 256)], buf, sem)

    def idx_wait(buf, sem):
        pltpu.make_async_copy(xt.at[0, pl.ds(0, 256)], buf, sem).wait()

    def gather_start(ibuf, rbuf, sem):
        pltpu.async_copy(lut_lin.at[ibuf], rbuf, sem)

    def gather_wait(ibuf, rbuf, sem):
        pltpu.make_async_copy(lut_lin.at[ibuf], rbuf, sem).wait()

    def out_start(u, obuf, sem):
        s = u // 16
        q = lax.rem(u, 16)
        for g in range(8):
            pltpu.async_copy(obuf.at[g], out5.at[s, g, pl.ds(q * 2, 2)], sem)

    def out_wait(obuf, sem):
        for g in range(8):
            pltpu.make_async_copy(obuf.at[g], out5.at[0, g, pl.ds(0, 2)], sem).wait()

    def transform(rbuf, obuf):
        @plsc.parallel_loop(0, 64, unroll=2)
        def _(c):
            g = c // 8
            cc = lax.rem(c, 8)
            col = jnp.broadcast_to(c, (16,)).astype(jnp.int32)
            for t2 in range(2):
                for m in range(8):
                    row_idx = iota + (128 * t2 + 16 * m)
                    v = plsc.load_gather(rbuf, [row_idx, col])
                    obuf[g, t2, cc, pl.ds(16 * m, 16)] = v

    pltpu.sync_copy(xt.at[u0 // 16, pl.ds(lax.rem(u0, 16) * 256, 256)], i0)
    gather_start(i0, r0, gsem0)
    idx_start(u0 + 1, i1, xsem1)

    def loop(i, carry):
        u = u0 + i
        b = lax.rem(i, 2)
        for b_s in range(2):
            @pl.when(b == b_s)
            def _():
                ib, rb, ob = ibufs[b_s], rows[b_s], obufs[b_s]
                nib, nrb = ibufs[1 - b_s], rows[1 - b_s]

                @pl.when(i + 1 < K2_UNITS_PER_W)
                def _():
                    idx_wait(nib, xsems[1 - b_s])
                    gather_start(nib, nrb, gsems[1 - b_s])

                gather_wait(ib, rb, gsems[b_s])

                @pl.when(i + 2 < K2_UNITS_PER_W)
                def _():
                    idx_start(u + 2, ib, xsems[b_s])

                @pl.when(i >= 2)
                def _():
                    out_wait(ob, osems[b_s])

                transform(rb, ob)
                out_start(u, ob, osems[b_s])
        return carry

    lax.fori_loop(0, K2_UNITS_PER_W, loop, 0)

    for i_tail in (K2_UNITS_PER_W - 2, K2_UNITS_PER_W - 1):
        b_t = i_tail % 2
        out_wait(obufs[b_t], osems[b_t])


@jax.jit
def _run(xt, lut_t):
    mesh = plsc.VectorSubcoreMesh(core_axis_name="c", subcore_axis_name="s")
    k1 = pl.kernel(
        _k1_body,
        out_type=jax.ShapeDtypeStruct((V // 2, 128), jnp.float32),
        mesh=mesh,
        compiler_params=pltpu.CompilerParams(
            use_tc_tiling_on_sc=True, needs_layout_passes=False
        ),
        scratch_types=[
            pltpu.VMEM((64, 128), jnp.float32),
            pltpu.VMEM((64, 128), jnp.float32),
            pltpu.VMEM((64, 128), jnp.float32),
            pltpu.VMEM((64, 128), jnp.float32),
            pltpu.VMEM((64, 64), jnp.float32),
            pltpu.VMEM((32, 128), jnp.float32),
            pltpu.SemaphoreType.DMA,
            pltpu.SemaphoreType.DMA,
            pltpu.SemaphoreType.DMA,
            pltpu.SemaphoreType.DMA,
        ],
    )
    o = k1(lut_t)
    lut_lin = o.reshape(V, D)
    k2 = pl.kernel(
        _k2_body,
        out_type=jax.ShapeDtypeStruct((200, 8, 32, 8, 128), jnp.float32),
        mesh=mesh,
        compiler_params=pltpu.CompilerParams(
            use_tc_tiling_on_sc=False, needs_layout_passes=False
        ),
        scratch_types=[
            pltpu.VMEM((256,), jnp.int32),
            pltpu.VMEM((256,), jnp.int32),
            pltpu.VMEM((256, 64), jnp.float32),
            pltpu.VMEM((256, 64), jnp.float32),
            pltpu.VMEM((8, 2, 8, 128), jnp.float32),
            pltpu.VMEM((8, 2, 8, 128), jnp.float32),
            pltpu.SemaphoreType.DMA,
            pltpu.SemaphoreType.DMA,
            pltpu.SemaphoreType.DMA,
            pltpu.SemaphoreType.DMA,
            pltpu.SemaphoreType.DMA,
            pltpu.SemaphoreType.DMA,
        ],
    )
    return k2(xt, lut_lin)


def kernel(x, lut):
    out5 = _run(x.T, lut.T)
    return out5.transpose(2, 4, 0, 1, 3).reshape(4096, 200, 64)


# speedup vs baseline: 3.0367x; 1.4975x over previous
import jax
import jax.numpy as jnp
from jax import lax
from jax.experimental import pallas as pl
from jax.experimental.pallas import tpu as pltpu
from jax.experimental.pallas import tpu_sc as plsc

D = 64
V = 1000000
SCALE = 8.0

NUM_CORES = 2
NUM_SUBCORES = 16
NW = NUM_CORES * NUM_SUBCORES

T_FULL = V // 128
K1_ITERS = (T_FULL + 1 + NW - 1) // NW


def _k1_body(lut_t, o_hbm, in0, in1, out0, out1, tailbuf, tailout,
             isem0, isem1, osem0, osem1):
    wid = lax.axis_index("s") * NUM_CORES + lax.axis_index("c")
    ins = (in0, in1)
    outs = (out0, out1)
    isems = (isem0, isem1)
    osems = (osem0, osem1)
    iota = lax.iota(jnp.int32, 16)

    def in_start(t, buf, sem):
        pltpu.async_copy(
            lut_t.at[:, pl.ds(t * 128, 128)], buf.at[:, pl.ds(0, 128)], sem
        )

    def in_wait(buf, sem):
        pltpu.make_async_copy(
            lut_t.at[:, pl.ds(0, 128)], buf.at[:, pl.ds(0, 128)], sem
        ).wait()

    def out_start(t, buf, sem):
        pltpu.async_copy(buf, o_hbm.at[pl.ds(t * 64, 64)], sem)

    def out_wait(buf, sem):
        pltpu.make_async_copy(buf, o_hbm.at[pl.ds(0, 64)], sem).wait()

    def transform(src, dst, nrows):
        @plsc.parallel_loop(0, nrows, unroll=4)
        def _(j):
            c0 = jnp.broadcast_to(2 * j, (16,)).astype(jnp.int32)
            c1 = c0 + 1
            for k in range(8):
                row_idx = iota + 16 * (k % 4)
                col = c0 if k < 4 else c1
                v = plsc.load_gather(src, [row_idx, col])
                dst[j, pl.ds(16 * k, 16)] = v * SCALE

    @pl.when(wid < T_FULL)
    def _():
        in_start(wid, in0, isem0)

    def loop(i, carry):
        t = wid + NW * i
        b = lax.rem(i, 2)
        t_next = t + NW

        @pl.when(t < T_FULL)
        def _():
            for b_s in range(2):
                @pl.when(b == b_s)
                def _():
                    @pl.when(t_next < T_FULL)
                    def _():
                        in_start(t_next, ins[1 - b_s], isems[1 - b_s])

                    in_wait(ins[b_s], isems[b_s])

                    @pl.when(i >= 2)
                    def _():
                        out_wait(outs[b_s], osems[b_s])

                    transform(ins[b_s], outs[b_s], 64)
                    out_start(t, outs[b_s], osems[b_s])

        @pl.when(t == T_FULL)
        def _():
            pltpu.sync_copy(lut_t.at[:, pl.ds(T_FULL * 128, 64)], tailbuf)
            transform(tailbuf, tailout, 32)
            pltpu.sync_copy(tailout, o_hbm.at[pl.ds(T_FULL * 64, 32)])

        return carry

    lax.fori_loop(0, K1_ITERS, loop, 0)

    n_full = (T_FULL - wid + NW - 1) // NW

    @pl.when(n_full >= 1)
    def _():
        blast = lax.rem(n_full - 1, 2)
        for b_s in range(2):
            @pl.when(blast == b_s)
            def _():
                out_wait(outs[b_s], osems[b_s])

    @pl.when(n_full >= 2)
    def _():
        bprev = lax.rem(n_full - 2, 2)
        for b_s in range(2):
            @pl.when(bprev == b_s)
            def _():
                out_wait(outs[b_s], osems[b_s])


K2_UNITS_PER_W = 3200 // NW


def _k2_body(xt, lut_lin, out5, i0, i1, r0, r1, rpad, ob0, ob1,
             xsem0, xsem1, gsem0, gsem1, osem0, osem1):
    wid = lax.axis_index("s") * NUM_CORES + lax.axis_index("c")
    u0 = wid * K2_UNITS_PER_W
    ibufs = (i0, i1)
    rows = (r0, r1)
    obufs = (ob0, ob1)
    xsems = (xsem0, xsem1)
    gsems = (gsem0, gsem1)
    osems = (osem0, osem1)
    iota = lax.iota(jnp.int32, 16)

    def idx_start(u, buf, sem):
        s = u // 16
        q = lax.rem(u, 16)
        pltpu.async_copy(xt.at[s, pl.ds(q * 256, 256)], buf, sem)

    def idx_wait(buf, sem):
        pltpu.make_async_copy(xt.at[0, pl.ds(0, 256)], buf, sem).wait()

    def gather_start(ibuf, rbuf, sem):
        pltpu.async_copy(lut_lin.at[ibuf], rbuf, sem)

    def gather_wait(ibuf, rbuf, sem):
        pltpu.make_async_copy(lut_lin.at[ibuf], rbuf, sem).wait()

    def out_start(u, obuf, sem):
        s = u // 16
        q = lax.rem(u, 16)
        for g in range(8):
            pltpu.async_copy(obuf.at[g], out5.at[s, g, pl.ds(q * 2, 2)], sem)

    def out_wait(obuf, sem):
        for g in range(8):
            pltpu.make_async_copy(obuf.at[g], out5.at[0, g, pl.ds(0, 2)], sem).wait()

    def transform(rbuf, obuf):
        @plsc.parallel_loop(0, 256, unroll=8)
        def _(r):
            for k in range(4):
                rpad[r, pl.ds(16 * k, 16)] = rbuf[r, pl.ds(16 * k, 16)]

        @plsc.parallel_loop(0, 64, unroll=2)
        def _(c):
            g = c // 8
            cc = lax.rem(c, 8)
            col = jnp.broadcast_to(c, (16,)).astype(jnp.int32)
            for t2 in range(2):
                for m in range(8):
                    row_idx = iota + (128 * t2 + 16 * m)
                    v = plsc.load_gather(rpad, [row_idx, col])
                    obuf[g, t2, cc, pl.ds(16 * m, 16)] = v

    pltpu.sync_copy(xt.at[u0 // 16, pl.ds(lax.rem(u0, 16) * 256, 256)], i0)
    gather_start(i0, r0, gsem0)
    idx_start(u0 + 1, i1, xsem1)

    def loop(i, carry):
        u = u0 + i
        b = lax.rem(i, 2)
        for b_s in range(2):
            @pl.when(b == b_s)
            def _():
                ib, rb, ob = ibufs[b_s], rows[b_s], obufs[b_s]
                nib, nrb = ibufs[1 - b_s], rows[1 - b_s]

                @pl.when(i + 1 < K2_UNITS_PER_W)
                def _():
                    idx_wait(nib, xsems[1 - b_s])
                    gather_start(nib, nrb, gsems[1 - b_s])

                gather_wait(ib, rb, gsems[b_s])

                @pl.when(i + 2 < K2_UNITS_PER_W)
                def _():
                    idx_start(u + 2, ib, xsems[b_s])

                @pl.when(i >= 2)
                def _():
                    out_wait(ob, osems[b_s])

                transform(rb, ob)
                out_start(u, ob, osems[b_s])
        return carry

    lax.fori_loop(0, K2_UNITS_PER_W, loop, 0)

    for i_tail in (K2_UNITS_PER_W - 2, K2_UNITS_PER_W - 1):
        b_t = i_tail % 2
        out_wait(obufs[b_t], osems[b_t])


@jax.jit
def _run(xt, lut_t):
    mesh = plsc.VectorSubcoreMesh(core_axis_name="c", subcore_axis_name="s")
    k1 = pl.kernel(
        _k1_body,
        out_type=jax.ShapeDtypeStruct((V // 2, 128), jnp.float32),
        mesh=mesh,
        compiler_params=pltpu.CompilerParams(
            use_tc_tiling_on_sc=True, needs_layout_passes=False
        ),
        scratch_types=[
            pltpu.VMEM((64, 129), jnp.float32),
            pltpu.VMEM((64, 129), jnp.float32),
            pltpu.VMEM((64, 128), jnp.float32),
            pltpu.VMEM((64, 128), jnp.float32),
            pltpu.VMEM((64, 64), jnp.float32),
            pltpu.VMEM((32, 128), jnp.float32),
            pltpu.SemaphoreType.DMA,
            pltpu.SemaphoreType.DMA,
            pltpu.SemaphoreType.DMA,
            pltpu.SemaphoreType.DMA,
        ],
    )
    o = k1(lut_t)
    lut_lin = o.reshape(V, D)
    k2 = pl.kernel(
        _k2_body,
        out_type=jax.ShapeDtypeStruct((200, 8, 32, 8, 128), jnp.float32),
        mesh=mesh,
        compiler_params=pltpu.CompilerParams(
            use_tc_tiling_on_sc=False, needs_layout_passes=False
        ),
        scratch_types=[
            pltpu.VMEM((256,), jnp.int32),
            pltpu.VMEM((256,), jnp.int32),
            pltpu.VMEM((256, 64), jnp.float32),
            pltpu.VMEM((256, 64), jnp.float32),
            pltpu.VMEM((256, 65), jnp.float32),
            pltpu.VMEM((8, 2, 8, 128), jnp.float32),
            pltpu.VMEM((8, 2, 8, 128), jnp.float32),
            pltpu.SemaphoreType.DMA,
            pltpu.SemaphoreType.DMA,
            pltpu.SemaphoreType.DMA,
            pltpu.SemaphoreType.DMA,
            pltpu.SemaphoreType.DMA,
            pltpu.SemaphoreType.DMA,
        ],
    )
    return k2(xt, lut_lin)


def kernel(x, lut):
    out5 = _run(x.T, lut.T)
    return out5.transpose(2, 4, 0, 1, 3).reshape(4096, 200, 64)
